# baseline (speedup 1.0000x reference)
import functools

import jax
import jax.numpy as jnp
from jax import lax
from jax.experimental import pallas as pl
from jax.experimental.pallas import tpu as pltpu
from jax.experimental.pallas import tpu_sc as plsc

VOCAB = 50000
HIDDEN = 1024
B, S = 4, 2048
N_TOK = B * S
EPS = 1e-12

NC, NS, L = 2, 16, 16
NW = NC * NS
TPW = N_TOK // NW
CK = 64
NCHUNK = TPW // CK
HC = HIDDEN // L
ST = 8

_mesh = plsc.VectorSubcoreMesh(
    core_axis_name="c", subcore_axis_name="s", num_cores=NC, num_subcores=NS
)


def _bcast_lane(vec, lane):
    return vec.at[jnp.full((L,), lane, jnp.int32)].get(mode="promise_in_bounds")


def _rsqrt(v):
    bits = lax.bitcast_convert_type(v, jnp.int32)
    y = lax.bitcast_convert_type(jnp.int32(0x5F3759DF) - (bits >> 1),
                                 jnp.float32)
    for _ in range(3):
        y = y * (1.5 - 0.5 * v * y * y)
    return y


@functools.partial(
    pl.kernel,
    out_type=jax.ShapeDtypeStruct((N_TOK, HIDDEN), jnp.float32),
    mesh=_mesh,
    scratch_types=[
        pltpu.VMEM((CK,), jnp.int32),
        pltpu.VMEM((CK,), jnp.float32),
        pltpu.VMEM((CK, HIDDEN), jnp.float32),
        pltpu.VMEM((2, HIDDEN), jnp.float32),
        pltpu.VMEM((HIDDEN,), jnp.float32),
        pltpu.VMEM((HIDDEN,), jnp.float32),
        pltpu.VMEM((HIDDEN,), jnp.float32),
        pltpu.SemaphoreType.DMA,
    ],
    compiler_params=pltpu.CompilerParams(needs_layout_passes=False),
)
def _emb_ln_kernel(word_hbm, ids_hbm, ttf_hbm, tt_hbm, gamma_hbm, beta_hbm,
                   out_hbm, idx_v, ttf_v, buf_v, tt_v, d_v, gamma_v, beta_v,
                   sem):
    cid = lax.axis_index("c")
    sid = lax.axis_index("s")
    wid = sid * NC + cid
    tok0 = wid * TPW

    pltpu.sync_copy(gamma_hbm, gamma_v)
    pltpu.sync_copy(beta_hbm, beta_v)
    pltpu.sync_copy(tt_hbm, tt_v)

    def dchunk(ci, _):
        sl = pl.ds(ci * L, L)
        d_v[sl] = tt_v[1, sl] - tt_v[0, sl]
        return 0

    lax.fori_loop(0, HC, dchunk, 0)

    inv_h = jnp.float32(1.0 / HIDDEN)
    zero = jnp.zeros((L,), jnp.float32)

    def kc_body(kc, _):
        base = tok0 + kc * CK
        pltpu.sync_copy(ids_hbm.at[pl.ds(base, CK)], idx_v)
        pltpu.sync_copy(ttf_hbm.at[pl.ds(base, CK)], ttf_v)
        pltpu.async_copy(word_hbm.at[idx_v], buf_v, sem).wait()

        def grp_body(g, _):
            tvec = ttf_v[pl.ds(g * L, L)]
            for h in range(L // ST):
                i0 = g * L + h * ST
                ts = [_bcast_lane(tvec, h * ST + j) for j in range(ST)]

                def pass_a(ci, carry):
                    sl = pl.ds(ci * L, L)
                    tt0c = tt_v[0, sl]
                    dc = d_v[sl]
                    out = []
                    for j in range(ST):
                        x = buf_v[i0 + j, sl]
                        x = x + (tt0c + ts[j] * dc)
                        buf_v[i0 + j, sl] = x
                        out.append(carry[2 * j] + x)
                        out.append(carry[2 * j + 1] + x * x)
                    return tuple(out)

                accs = lax.fori_loop(0, HC, pass_a, (zero,) * (2 * ST),
                                     unroll=4)

                mvs, ys = [], []
                for j in range(ST):
                    mean = jnp.sum(accs[2 * j]) * inv_h
                    var = jnp.sum(accs[2 * j + 1]) * inv_h - mean * mean
                    mvs.append(jnp.full((L,), mean, jnp.float32))
                    ys.append(_rsqrt(jnp.full((L,), var + EPS, jnp.float32)))

                def pass_b(ci, _):
                    sl = pl.ds(ci * L, L)
                    gc = gamma_v[sl]
                    bc = beta_v[sl]
                    for j in range(ST):
                        xh = (buf_v[i0 + j, sl] - mvs[j]) * ys[j]
                        buf_v[i0 + j, sl] = xh * gc + bc
                    return 0

                lax.fori_loop(0, HC, pass_b, 0, unroll=4)
            return 0

        lax.fori_loop(0, CK // L, grp_body, 0)
        pltpu.sync_copy(buf_v, out_hbm.at[pl.ds(base, CK)])
        return 0

    lax.fori_loop(0, NCHUNK, kc_body, 0)


def kernel(input_ids, token_type_ids, word_embeddings, token_type_embeddings,
           ln_gamma, ln_beta):
    ids = input_ids.reshape(-1).astype(jnp.int32)
    ttf = token_type_ids.reshape(-1).astype(jnp.float32)
    out = _emb_ln_kernel(word_embeddings, ids, ttf, token_type_embeddings,
                         ln_gamma, ln_beta)
    return out.reshape(B, S, HIDDEN)


# speedup vs baseline: 1.2477x; 1.2477x over previous
import functools

import jax
import jax.numpy as jnp
from jax import lax
from jax.experimental import pallas as pl
from jax.experimental.pallas import tpu as pltpu
from jax.experimental.pallas import tpu_sc as plsc

VOCAB = 50000
HIDDEN = 1024
B, S = 4, 2048
N_TOK = B * S
EPS = 1e-12

NC, NS, L = 2, 16, 16
NW = NC * NS
TPW = N_TOK // NW
CK = 32
NCHUNK = TPW // CK
HC = HIDDEN // L
ST = 8
NB = 3

_mesh = plsc.VectorSubcoreMesh(
    core_axis_name="c", subcore_axis_name="s", num_cores=NC, num_subcores=NS
)


def _bcast_lane(vec, lane):
    return vec.at[jnp.full((L,), lane, jnp.int32)].get(mode="promise_in_bounds")


def _rsqrt(v):
    bits = lax.bitcast_convert_type(v, jnp.int32)
    y = lax.bitcast_convert_type(jnp.int32(0x5F3759DF) - (bits >> 1),
                                 jnp.float32)
    for _ in range(3):
        y = y * (1.5 - 0.5 * v * y * y)
    return y


@functools.partial(
    pl.kernel,
    out_type=jax.ShapeDtypeStruct((N_TOK, HIDDEN), jnp.float32),
    mesh=_mesh,
    scratch_types=[
        pltpu.VMEM((NB, CK), jnp.int32),
        pltpu.VMEM((NB, CK), jnp.float32),
        pltpu.VMEM((NB, CK, HIDDEN), jnp.float32),
        pltpu.VMEM((2, HIDDEN), jnp.float32),
        pltpu.VMEM((HIDDEN,), jnp.float32),
        pltpu.VMEM((HIDDEN,), jnp.float32),
        pltpu.VMEM((HIDDEN,), jnp.float32),
        pltpu.SemaphoreType.DMA((NB,)),
        pltpu.SemaphoreType.DMA((NB,)),
    ],
    compiler_params=pltpu.CompilerParams(needs_layout_passes=False),
)
def _emb_ln_kernel(word_hbm, ids_hbm, ttf_hbm, tt_hbm, gamma_hbm, beta_hbm,
                   out_hbm, idx_v, ttf_v, buf_v, tt_v, d_v, gamma_v, beta_v,
                   gsem, wsem):
    cid = lax.axis_index("c")
    sid = lax.axis_index("s")
    wid = sid * NC + cid
    tok0 = wid * TPW

    pltpu.sync_copy(gamma_hbm, gamma_v)
    pltpu.sync_copy(beta_hbm, beta_v)
    pltpu.sync_copy(tt_hbm, tt_v)

    def dchunk(ci, _):
        sl = pl.ds(ci * L, L)
        d_v[sl] = tt_v[1, sl] - tt_v[0, sl]
        return 0

    lax.fori_loop(0, HC, dchunk, 0)

    inv_h = jnp.float32(1.0 / HIDDEN)
    zero = jnp.zeros((L,), jnp.float32)

    for p in range(2):
        pbase = tok0 + p * CK
        pltpu.sync_copy(ids_hbm.at[pl.ds(pbase, CK)], idx_v.at[p])
        pltpu.sync_copy(ttf_hbm.at[pl.ds(pbase, CK)], ttf_v.at[p])
        pltpu.async_copy(word_hbm.at[idx_v.at[p]], buf_v.at[p], gsem.at[p])

    def kc_body(kc, _):
        b = lax.rem(kc, NB)
        base = tok0 + kc * CK
        pltpu.make_async_copy(word_hbm.at[idx_v.at[b]], buf_v.at[b],
                              gsem.at[b]).wait()

        def grp_body(g, _):
            tvec = ttf_v[b, pl.ds(g * L, L)]
            for h in range(L // ST):
                i0 = g * L + h * ST
                ts = [_bcast_lane(tvec, h * ST + j) for j in range(ST)]

                def pass_a(ci, carry):
                    sl = pl.ds(ci * L, L)
                    tt0c = tt_v[0, sl]
                    dc = d_v[sl]
                    out = []
                    for j in range(ST):
                        x = buf_v[b, i0 + j, sl]
                        x = x + (tt0c + ts[j] * dc)
                        buf_v[b, i0 + j, sl] = x
                        out.append(carry[2 * j] + x)
                        out.append(carry[2 * j + 1] + x * x)
                    return tuple(out)

                accs = lax.fori_loop(0, HC, pass_a, (zero,) * (2 * ST))

                mvs, ys = [], []
                for j in range(ST):
                    mean = jnp.sum(accs[2 * j]) * inv_h
                    var = jnp.sum(accs[2 * j + 1]) * inv_h - mean * mean
                    mvs.append(jnp.full((L,), mean, jnp.float32))
                    ys.append(_rsqrt(jnp.full((L,), var + EPS, jnp.float32)))

                def pass_b(ci, _):
                    sl = pl.ds(ci * L, L)
                    gc = gamma_v[sl]
                    bc = beta_v[sl]
                    for j in range(ST):
                        xh = (buf_v[b, i0 + j, sl] - mvs[j]) * ys[j]
                        buf_v[b, i0 + j, sl] = xh * gc + bc
                    return 0

                lax.fori_loop(0, HC, pass_b, 0)
            return 0

        lax.fori_loop(0, CK // L, grp_body, 0)
        pltpu.async_copy(buf_v.at[b], out_hbm.at[pl.ds(base, CK)], wsem.at[b])

        @pl.when(kc + 2 < NCHUNK)
        def _prefetch():
            bp = lax.rem(kc + 2, NB)
            nbase = tok0 + (kc + 2) * CK

            @pl.when(kc >= 1)
            def _drain():
                pltpu.make_async_copy(
                    buf_v.at[bp], out_hbm.at[pl.ds(nbase - NB * CK, CK)],
                    wsem.at[bp]).wait()

            pltpu.sync_copy(ids_hbm.at[pl.ds(nbase, CK)], idx_v.at[bp])
            pltpu.sync_copy(ttf_hbm.at[pl.ds(nbase, CK)], ttf_v.at[bp])
            pltpu.async_copy(word_hbm.at[idx_v.at[bp]], buf_v.at[bp],
                             gsem.at[bp])

        return 0

    lax.fori_loop(0, NCHUNK, kc_body, 0)

    for c in range(NCHUNK - NB, NCHUNK):
        s = c % NB
        pltpu.make_async_copy(buf_v.at[s], out_hbm.at[pl.ds(tok0 + c * CK, CK)],
                              wsem.at[s]).wait()


def kernel(input_ids, token_type_ids, word_embeddings, token_type_embeddings,
           ln_gamma, ln_beta):
    ids = input_ids.reshape(-1).astype(jnp.int32)
    ttf = token_type_ids.reshape(-1).astype(jnp.float32)
    out = _emb_ln_kernel(word_embeddings, ids, ttf, token_type_embeddings,
                         ln_gamma, ln_beta)
    return out.reshape(B, S, HIDDEN)


# speedup vs baseline: 1.4297x; 1.1459x over previous
import functools

import jax
import jax.numpy as jnp
from jax import lax
from jax.experimental import pallas as pl
from jax.experimental.pallas import tpu as pltpu
from jax.experimental.pallas import tpu_sc as plsc

VOCAB = 50000
HIDDEN = 1024
B, S = 4, 2048
N_TOK = B * S
EPS = 1e-12

NC, NS, L = 2, 16, 16
NW = NC * NS
TPW = N_TOK // NW
CK = 32
NCHUNK = TPW // CK
HC = HIDDEN // L
ST = 8
NB = 3

_mesh = plsc.VectorSubcoreMesh(
    core_axis_name="c", subcore_axis_name="s", num_cores=NC, num_subcores=NS
)


def _bcast_lane(vec, lane):
    return vec.at[jnp.full((L,), lane, jnp.int32)].get(mode="promise_in_bounds")


def _rsqrt(v):
    bits = lax.bitcast_convert_type(v, jnp.int32)
    y = lax.bitcast_convert_type(jnp.int32(0x5F3759DF) - (bits >> 1),
                                 jnp.float32)
    for _ in range(3):
        y = y * (1.5 - 0.5 * v * y * y)
    return y


@functools.partial(
    pl.kernel,
    out_type=jax.ShapeDtypeStruct((N_TOK, HIDDEN), jnp.float32),
    mesh=_mesh,
    scratch_types=[
        pltpu.VMEM((NB, CK), jnp.int32),
        pltpu.VMEM((NB, CK), jnp.float32),
        pltpu.VMEM((NB, CK, HIDDEN), jnp.float32),
        pltpu.VMEM((2, HIDDEN), jnp.float32),
        pltpu.VMEM((HIDDEN,), jnp.float32),
        pltpu.VMEM((HIDDEN,), jnp.float32),
        pltpu.VMEM((HIDDEN,), jnp.float32),
        pltpu.SemaphoreType.DMA((NB,)),
        pltpu.SemaphoreType.DMA((NB,)),
    ],
    compiler_params=pltpu.CompilerParams(needs_layout_passes=False),
)
def _emb_ln_kernel(word_hbm, ids_hbm, ttf_hbm, tt_hbm, gamma_hbm, beta_hbm,
                   out_hbm, idx_v, ttf_v, buf_v, tt_v, d_v, gamma_v, beta_v,
                   gsem, wsem):
    cid = lax.axis_index("c")
    sid = lax.axis_index("s")
    wid = sid * NC + cid
    tok0 = wid * TPW

    pltpu.sync_copy(gamma_hbm, gamma_v)
    pltpu.sync_copy(beta_hbm, beta_v)
    pltpu.sync_copy(tt_hbm, tt_v)

    def dchunk(ci, _):
        sl = pl.ds(ci * L, L)
        d_v[sl] = tt_v[1, sl] - tt_v[0, sl]
        return 0

    lax.fori_loop(0, HC, dchunk, 0)

    inv_h = jnp.float32(1.0 / HIDDEN)
    zero = jnp.zeros((L,), jnp.float32)

    for p in range(2):
        pbase = tok0 + p * CK
        pltpu.sync_copy(ids_hbm.at[pl.ds(pbase, CK)], idx_v.at[p])
        pltpu.sync_copy(ttf_hbm.at[pl.ds(pbase, CK)], ttf_v.at[p])
        pltpu.async_copy(word_hbm.at[idx_v.at[p]], buf_v.at[p], gsem.at[p])

    def kc_body(kc, _):
        b = lax.rem(kc, NB)
        base = tok0 + kc * CK
        pltpu.make_async_copy(word_hbm.at[idx_v.at[b]], buf_v.at[b],
                              gsem.at[b]).wait()

        def grp_body(g, _):
            tvec = ttf_v[b, pl.ds(g * L, L)]
            for h in range(L // ST):
                i0 = g * L + h * ST
                ts = [_bcast_lane(tvec, h * ST + j) for j in range(ST)]

                @plsc.parallel_loop(0, HC, unroll=2, carry=(zero,) * (2 * ST))
                def pass_a(ci, carry):
                    sl = pl.ds(ci * L, L)
                    tt0c = tt_v[0, sl]
                    dc = d_v[sl]
                    out = []
                    for j in range(ST):
                        x = buf_v[b, i0 + j, sl]
                        x = x + (tt0c + ts[j] * dc)
                        buf_v[b, i0 + j, sl] = x
                        out.append(carry[2 * j] + x)
                        out.append(carry[2 * j + 1] + x * x)
                    return tuple(out)

                accs = pass_a

                mvs, ys = [], []
                for j in range(ST):
                    mean = jnp.sum(accs[2 * j]) * inv_h
                    var = jnp.sum(accs[2 * j + 1]) * inv_h - mean * mean
                    mvs.append(jnp.full((L,), mean, jnp.float32))
                    ys.append(_rsqrt(jnp.full((L,), var + EPS, jnp.float32)))

                @plsc.parallel_loop(0, HC, unroll=2)
                def pass_b(ci):
                    sl = pl.ds(ci * L, L)
                    gc = gamma_v[sl]
                    bc = beta_v[sl]
                    for j in range(ST):
                        xh = (buf_v[b, i0 + j, sl] - mvs[j]) * ys[j]
                        buf_v[b, i0 + j, sl] = xh * gc + bc
            return 0

        lax.fori_loop(0, CK // L, grp_body, 0)
        pltpu.async_copy(buf_v.at[b], out_hbm.at[pl.ds(base, CK)], wsem.at[b])

        @pl.when(kc + 2 < NCHUNK)
        def _prefetch():
            bp = lax.rem(kc + 2, NB)
            nbase = tok0 + (kc + 2) * CK

            @pl.when(kc >= 1)
            def _drain():
                pltpu.make_async_copy(
                    buf_v.at[bp], out_hbm.at[pl.ds(nbase - NB * CK, CK)],
                    wsem.at[bp]).wait()

            pltpu.sync_copy(ids_hbm.at[pl.ds(nbase, CK)], idx_v.at[bp])
            pltpu.sync_copy(ttf_hbm.at[pl.ds(nbase, CK)], ttf_v.at[bp])
            pltpu.async_copy(word_hbm.at[idx_v.at[bp]], buf_v.at[bp],
                             gsem.at[bp])

        return 0

    lax.fori_loop(0, NCHUNK, kc_body, 0)

    for c in range(NCHUNK - NB, NCHUNK):
        s = c % NB
        pltpu.make_async_copy(buf_v.at[s], out_hbm.at[pl.ds(tok0 + c * CK, CK)],
                              wsem.at[s]).wait()


def kernel(input_ids, token_type_ids, word_embeddings, token_type_embeddings,
           ln_gamma, ln_beta):
    ids = input_ids.reshape(-1).astype(jnp.int32)
    ttf = token_type_ids.reshape(-1).astype(jnp.float32)
    out = _emb_ln_kernel(word_embeddings, ids, ttf, token_type_embeddings,
                         ln_gamma, ln_beta)
    return out.reshape(B, S, HIDDEN)
